# baseline (speedup 1.0000x reference)
import functools

import jax
import jax.numpy as jnp
from jax.experimental import pallas as pl
from jax.experimental.pallas import tpu as pltpu

MAX_DIST = 32
TABLE_SIZE = 2 * MAX_DIST + 1
VALS = MAX_DIST + 1
K = 128
NBUF = 4


def _bias_kernel(o_all_ref, m_ref, out_ref, scratch, sems, *, nh, nsteps):
    b = pl.program_id(0)
    h = pl.program_id(1)
    step = b * nh + h
    slot = jax.lax.rem(step, NBUF)

    @pl.when(step >= NBUF)
    def _wait_prev():
        so = step - NBUF
        pltpu.make_async_copy(
            scratch.at[slot], out_ref.at[so // nh, so - (so // nh) * nh],
            sems.at[slot]).wait()

    of = o_all_ref[0]
    m = m_ref[0].astype(jnp.bfloat16)
    a = jnp.dot(of, m, preferred_element_type=jnp.float32)
    out = jax.lax.dot_general(
        a.astype(jnp.bfloat16), of, (((1,), (1,)), ((), ())),
        preferred_element_type=jnp.float32)
    scratch[slot] = out
    pltpu.make_async_copy(
        scratch.at[slot], out_ref.at[b, h], sems.at[slot]).start()

    @pl.when(step == nsteps - 1)
    def _drain():
        for k in range(NBUF):
            so = nsteps - NBUF + k
            pltpu.make_async_copy(
                scratch.at[so % NBUF], out_ref.at[so // nh, so % nh],
                sems.at[so % NBUF]).wait()


@functools.partial(jax.jit, static_argnames=())
def kernel(positions, rel_bias_d, rel_bias_h, rel_bias_w):
    B, N, _ = positions.shape
    H = rel_bias_d.shape[1]

    pos = jnp.clip(positions.astype(jnp.int32), 0, MAX_DIST)
    ks = jnp.arange(K, dtype=jnp.int32)
    onehot = ((pos[:, :, 0, None] == ks)
              | (pos[:, :, 1, None] + VALS == ks)
              | (pos[:, :, 2, None] + 2 * VALS == ks)).astype(jnp.bfloat16)

    u = jnp.arange(VALS, dtype=jnp.int32)
    duv = u[:, None] - u[None, :] + MAX_DIST
    md = rel_bias_d[duv].transpose(2, 0, 1)
    mh = rel_bias_h[duv].transpose(2, 0, 1)
    mw = rel_bias_w[duv].transpose(2, 0, 1)
    m = jnp.zeros((H, K, K), dtype=jnp.float32)
    m = m.at[:, 0:VALS, 0:VALS].set(md)
    m = m.at[:, VALS:2 * VALS, VALS:2 * VALS].set(mh)
    m = m.at[:, 2 * VALS:3 * VALS, 2 * VALS:3 * VALS].set(mw)

    grid = (B, H)
    out = pl.pallas_call(
        functools.partial(_bias_kernel, nh=H, nsteps=B * H),
        grid=grid,
        in_specs=[
            pl.BlockSpec((1, N, K), lambda b, h: (b, 0, 0)),
            pl.BlockSpec((1, K, K), lambda b, h: (h, 0, 0)),
        ],
        out_specs=pl.BlockSpec(memory_space=pl.ANY),
        out_shape=jax.ShapeDtypeStruct((B, H, N, N), jnp.float32),
        scratch_shapes=[
            pltpu.VMEM((NBUF, N, N), jnp.float32),
            pltpu.SemaphoreType.DMA((NBUF,)),
        ],
    )(onehot, m)
    return out
